# Initial kernel scaffold; baseline (speedup 1.0000x reference)
import jax
import jax.numpy as jnp
from jax.experimental import pallas as pl


def kernel(texts, emb_table, W, b):
    raise NotImplementedError("write your pallas kernel here")


# speedup vs baseline: 86.5473x; 86.5473x over previous
import functools

import jax
import jax.numpy as jnp
from jax import lax
from jax.experimental import pallas as pl
from jax.experimental.pallas import tpu as pltpu
from jax.experimental.pallas import tpu_sc as plsc

_LANES = 16
_NUM_CORES = 2
_NUM_SUBCORES = 16
_NUM_WORKERS = _NUM_CORES * _NUM_SUBCORES


def _fold_body(e_ref, w_ref, b_ref, o_ref):
    o_ref[...] = (
        jnp.dot(e_ref[...], w_ref[...], preferred_element_type=jnp.float32)
        + b_ref[...]
    )


def _fold_table(emb_table, W, b2d):
    V, D = emb_table.shape
    C = W.shape[1]
    RB = 1024
    return pl.pallas_call(
        _fold_body,
        grid=(pl.cdiv(V, RB),),
        in_specs=[
            pl.BlockSpec((RB, D), lambda i: (i, 0)),
            pl.BlockSpec((D, C), lambda i: (0, 0)),
            pl.BlockSpec((1, C), lambda i: (0, 0)),
        ],
        out_specs=pl.BlockSpec((RB, C), lambda i: (i, 0)),
        out_shape=jax.ShapeDtypeStruct((V, C), jnp.float32),
    )(emb_table, W, b2d)


def _make_bag_kernel(V, C, B, L):
    assert C == 2 and B % (_NUM_WORKERS * _LANES) == 0
    bags_per_worker = B // _NUM_WORKERS
    groups_per_worker = bags_per_worker // _LANES
    inv_l = 1.0 / float(L)

    mesh = plsc.VectorSubcoreMesh(core_axis_name="c", subcore_axis_name="s")

    @functools.partial(
        pl.kernel,
        out_type=jax.ShapeDtypeStruct((B * C,), jnp.float32),
        mesh=mesh,
        scratch_types=[
            pltpu.VMEM((V * C,), jnp.float32),
            pltpu.VMEM((_LANES * L,), jnp.int32),
            pltpu.VMEM((bags_per_worker * C,), jnp.float32),
        ],
        compiler_params=pltpu.CompilerParams(needs_layout_passes=False),
    )
    def bag_kernel(a_hbm, t_hbm, o_hbm, a_v, t_v, o_v):
        wid = lax.axis_index("s") * _NUM_CORES + lax.axis_index("c")
        base = wid * bags_per_worker
        pltpu.sync_copy(a_hbm, a_v)

        lanes = lax.iota(jnp.int32, _LANES)
        lane_off = lanes * L
        ones16 = jnp.ones((_LANES,), jnp.int32)
        zacc = jnp.zeros((_LANES,), jnp.float32)

        def group(g, carry):
            row0 = base + g * _LANES
            pltpu.sync_copy(t_hbm.at[pl.ds(row0 * L, _LANES * L)], t_v)

            def step(t, accs):
                acc0, acc1 = accs
                idx = plsc.load_gather(t_v, [lane_off + t])
                a_idx = idx * 2
                acc0 = acc0 + plsc.load_gather(a_v, [a_idx])
                acc1 = acc1 + plsc.load_gather(a_v, [a_idx + 1])
                return acc0, acc1

            acc0, acc1 = lax.fori_loop(0, L, step, (zacc, zacc), unroll=8)
            oflat = (g * _LANES + lanes) * 2
            plsc.store_scatter(o_v, [oflat], acc0 * inv_l)
            plsc.store_scatter(o_v, [oflat + ones16], acc1 * inv_l)
            return carry

        lax.fori_loop(0, groups_per_worker, group, 0)
        pltpu.sync_copy(o_v, o_hbm.at[pl.ds(base * C, bags_per_worker * C)])

    return bag_kernel


def kernel(texts, emb_table, W, b):
    V, D = emb_table.shape
    C = W.shape[1]
    B, L = texts.shape
    A = _fold_table(emb_table, W, b.reshape(1, C))
    bag = _make_bag_kernel(V, C, B, L)
    out_flat = bag(A.reshape(V * C), texts.astype(jnp.int32).reshape(B * L))
    return out_flat.reshape(B, C)


# speedup vs baseline: 117.2140x; 1.3543x over previous
import functools

import jax
import jax.numpy as jnp
from jax import lax
from jax.experimental import pallas as pl
from jax.experimental.pallas import tpu as pltpu
from jax.experimental.pallas import tpu_sc as plsc

_LANES = 16
_NUM_CORES = 2
_NUM_SUBCORES = 16
_NUM_WORKERS = _NUM_CORES * _NUM_SUBCORES


def _fold_body(e_ref, w_ref, b_ref, o_ref):
    at = lax.dot_general(
        w_ref[...], e_ref[...],
        dimension_numbers=(((0,), (1,)), ((), ())),
        preferred_element_type=jnp.float32,
    ) + b_ref[...]
    bits = lax.bitcast_convert_type(at.astype(jnp.bfloat16), jnp.uint16)
    u = bits.astype(jnp.uint32)
    packed = (u[1:2, :] << 16) | u[0:1, :]
    o_ref[...] = lax.bitcast_convert_type(packed, jnp.int32)


def _fold_table(emb_table, W, b2d):
    V, D = emb_table.shape
    C = W.shape[1]
    RB = 1024
    return pl.pallas_call(
        _fold_body,
        grid=(pl.cdiv(V, RB),),
        in_specs=[
            pl.BlockSpec((RB, D), lambda i: (i, 0)),
            pl.BlockSpec((D, C), lambda i: (0, 0)),
            pl.BlockSpec((C, 1), lambda i: (0, 0)),
        ],
        out_specs=pl.BlockSpec((1, RB), lambda i: (0, i)),
        out_shape=jax.ShapeDtypeStruct((1, V), jnp.int32),
    )(emb_table, W, b2d)


def _make_bag_kernel(V, C, B, L):
    assert C == 2 and B % (_NUM_WORKERS * _LANES) == 0
    bags_per_worker = B // _NUM_WORKERS
    n_chunks = 4
    chunk_bags = bags_per_worker // n_chunks
    groups_per_chunk = chunk_bags // _LANES
    inv_l = 1.0 / float(L)

    mesh = plsc.VectorSubcoreMesh(core_axis_name="c", subcore_axis_name="s")

    @functools.partial(
        pl.kernel,
        out_type=jax.ShapeDtypeStruct((B * C,), jnp.float32),
        mesh=mesh,
        scratch_types=[
            pltpu.VMEM((V,), jnp.int32),
            pltpu.VMEM((chunk_bags * L,), jnp.int32),
            pltpu.VMEM((chunk_bags * L,), jnp.int32),
            pltpu.VMEM((bags_per_worker * C,), jnp.float32),
            pltpu.SemaphoreType.DMA,
            pltpu.SemaphoreType.DMA,
            pltpu.SemaphoreType.DMA,
        ],
        compiler_params=pltpu.CompilerParams(needs_layout_passes=False),
    )
    def bag_kernel(a_hbm, t_hbm, o_hbm, a_v, t_v0, t_v1, o_v, sem0, sem1, sem_a):
        wid = lax.axis_index("s") * _NUM_CORES + lax.axis_index("c")
        base = wid * bags_per_worker

        a_cp = pltpu.async_copy(a_hbm.at[0, :], a_v, sem_a)

        t_bufs = (t_v0, t_v1)
        sems = (sem0, sem1)

        def start_chunk(ci):
            src = t_hbm.at[pl.ds((base + ci * chunk_bags) * L, chunk_bags * L)]
            return pltpu.async_copy(src, t_bufs[ci % 2], sems[ci % 2])

        pending = start_chunk(0)

        lanes = lax.iota(jnp.int32, _LANES)
        lane_off = lanes * L
        rot = lanes * 9
        ones16 = jnp.ones((_LANES,), jnp.int32)
        himask = jnp.full((_LANES,), -65536, jnp.int32)
        zacc = jnp.zeros((_LANES,), jnp.float32)

        a_cp.wait()

        for ci in range(n_chunks):
            pending.wait()
            if ci + 1 < n_chunks:
                pending = start_chunk(ci + 1)
            tbuf = t_bufs[ci % 2]

            def group(g, carry, tbuf=tbuf, ci=ci):
                goff = g * (_LANES * L) + lane_off

                def step(t, accs):
                    acc0, acc1 = accs
                    tt = t + rot
                    tt = jnp.where(tt >= L, tt - L, tt)
                    idx = plsc.load_gather(tbuf, [goff + tt])
                    w = plsc.load_gather(a_v, [idx])
                    lo = plsc.bitcast(w << 16, jnp.float32)
                    hi = plsc.bitcast(w & himask, jnp.float32)
                    return acc0 + lo, acc1 + hi

                acc0, acc1 = lax.fori_loop(0, L, step, (zacc, zacc), unroll=10)
                oflat = ((ci * groups_per_chunk + g) * _LANES + lanes) * 2
                plsc.store_scatter(o_v, [oflat], acc0 * inv_l)
                plsc.store_scatter(o_v, [oflat + ones16], acc1 * inv_l)
                return carry

            lax.fori_loop(0, groups_per_chunk, group, 0)

        pltpu.sync_copy(o_v, o_hbm.at[pl.ds(base * C, bags_per_worker * C)])

    return bag_kernel


def kernel(texts, emb_table, W, b):
    V, D = emb_table.shape
    C = W.shape[1]
    B, L = texts.shape
    A = _fold_table(emb_table, W, b.reshape(C, 1))
    bag = _make_bag_kernel(V, C, B, L)
    out_flat = bag(A, texts.astype(jnp.int32).reshape(B * L))
    return out_flat.reshape(B, C)


# speedup vs baseline: 137.9210x; 1.1767x over previous
import functools

import jax
import jax.numpy as jnp
from jax import lax
from jax.experimental import pallas as pl
from jax.experimental.pallas import tpu as pltpu
from jax.experimental.pallas import tpu_sc as plsc

_LANES = 16
_NUM_CORES = 2
_NUM_SUBCORES = 16
_NUM_WORKERS = _NUM_CORES * _NUM_SUBCORES


def _fold_body(e_ref, w_ref, b_ref, o_ref):
    at = lax.dot_general(
        w_ref[...], e_ref[...],
        dimension_numbers=(((0,), (1,)), ((), ())),
        preferred_element_type=jnp.float32,
    ) + b_ref[...]
    bits = lax.bitcast_convert_type(at.astype(jnp.bfloat16), jnp.uint16)
    u = bits.astype(jnp.uint32)
    packed = (u[1:2, :] << 16) | u[0:1, :]
    o_ref[...] = lax.bitcast_convert_type(packed, jnp.int32)


def _fold_table(emb_table, W, b2d):
    V, D = emb_table.shape
    C = W.shape[1]
    RB = 2048
    return pl.pallas_call(
        _fold_body,
        grid=(pl.cdiv(V, RB),),
        in_specs=[
            pl.BlockSpec((RB, D), lambda i: (i, 0)),
            pl.BlockSpec((D, C), lambda i: (0, 0)),
            pl.BlockSpec((C, 1), lambda i: (0, 0)),
        ],
        out_specs=pl.BlockSpec((1, RB), lambda i: (0, i)),
        out_shape=jax.ShapeDtypeStruct((1, V), jnp.int32),
    )(emb_table, W, b2d)


def _make_bag_kernel(V, C, B, L):
    assert C == 2 and B % (_NUM_WORKERS * _LANES) == 0
    bags_per_worker = B // _NUM_WORKERS
    n_chunks = 4
    chunk_bags = bags_per_worker // n_chunks
    groups_per_chunk = chunk_bags // _LANES
    inv_l = 1.0 / float(L)

    mesh = plsc.VectorSubcoreMesh(core_axis_name="c", subcore_axis_name="s")

    @functools.partial(
        pl.kernel,
        out_type=jax.ShapeDtypeStruct((B * C,), jnp.float32),
        mesh=mesh,
        scratch_types=[
            pltpu.VMEM((V,), jnp.int32),
            pltpu.VMEM((chunk_bags, L), jnp.int32),
            pltpu.VMEM((chunk_bags, L), jnp.int32),
            pltpu.VMEM((bags_per_worker * C,), jnp.float32),
            pltpu.SemaphoreType.DMA,
            pltpu.SemaphoreType.DMA,
            pltpu.SemaphoreType.DMA,
        ],
        compiler_params=pltpu.CompilerParams(needs_layout_passes=False),
    )
    def bag_kernel(a_hbm, t_hbm, o_hbm, a_v, t_v0, t_v1, o_v, sem0, sem1, sem_a):
        wid = lax.axis_index("s") * _NUM_CORES + lax.axis_index("c")
        base = wid * bags_per_worker

        a_cp = pltpu.async_copy(a_hbm.at[0, :], a_v, sem_a)

        t_bufs = (t_v0, t_v1)
        sems = (sem0, sem1)

        def start_chunk(ci):
            src = t_hbm.at[pl.ds(base + ci * chunk_bags, chunk_bags), :]
            return pltpu.async_copy(src, t_bufs[ci % 2], sems[ci % 2])

        pending = start_chunk(0)

        lanes = lax.iota(jnp.int32, _LANES)
        rot = lanes * 9
        zeros16 = jnp.zeros((_LANES,), jnp.int32)
        ones16 = jnp.ones((_LANES,), jnp.int32)
        himask = jnp.full((_LANES,), -65536, jnp.int32)
        zacc = jnp.zeros((_LANES,), jnp.float32)

        a_cp.wait()

        for ci in range(n_chunks):
            pending.wait()
            if ci + 1 < n_chunks:
                pending = start_chunk(ci + 1)
            tbuf = t_bufs[ci % 2]

            def group(g, carry, tbuf=tbuf, ci=ci):
                rows = g * _LANES + lanes

                def step(t, accs):
                    acc0, acc1 = accs
                    tt = t + rot
                    tt = jnp.where(tt >= L, tt - L, tt)
                    idx = plsc.load_gather(tbuf, [rows, tt])
                    w = plsc.load_gather(a_v, [idx])
                    lo = plsc.bitcast(w << 16, jnp.float32)
                    hi = plsc.bitcast(w & himask, jnp.float32)
                    return acc0 + lo, acc1 + hi

                acc0, acc1 = lax.fori_loop(0, L, step, (zacc, zacc), unroll=10)
                oflat = ((ci * groups_per_chunk + g) * _LANES + lanes) * 2
                plsc.store_scatter(o_v, [oflat], acc0 * inv_l)
                plsc.store_scatter(o_v, [oflat + ones16], acc1 * inv_l)
                return carry

            lax.fori_loop(0, groups_per_chunk, group, 0)

        pltpu.sync_copy(o_v, o_hbm.at[pl.ds(base * C, bags_per_worker * C)])

    return bag_kernel


def kernel(texts, emb_table, W, b):
    V, D = emb_table.shape
    C = W.shape[1]
    B, L = texts.shape
    A = _fold_table(emb_table, W, b.reshape(C, 1))
    bag = _make_bag_kernel(V, C, B, L)
    out_flat = bag(A, texts.astype(jnp.int32))
    return out_flat.reshape(B, C)


# speedup vs baseline: 347.2350x; 2.5176x over previous
import functools

import jax
import jax.numpy as jnp
from jax import lax
from jax.experimental import pallas as pl
from jax.experimental.pallas import tpu as pltpu
from jax.experimental.pallas import tpu_sc as plsc

_LANES = 16
_NUM_CORES = 2
_NUM_SUBCORES = 16
_NUM_WORKERS = _NUM_CORES * _NUM_SUBCORES


def _fold_body(et_ref, wt_ref, b_ref, o_ref):
    at = lax.dot_general(
        wt_ref[...], et_ref[...],
        dimension_numbers=(((1,), (0,)), ((), ())),
        preferred_element_type=jnp.float32,
    ) + b_ref[...]
    bits = lax.bitcast_convert_type(at.astype(jnp.bfloat16), jnp.uint16)
    u = bits.astype(jnp.uint32)
    packed = (u[1:2, :] << 16) | u[0:1, :]
    o_ref[...] = lax.bitcast_convert_type(packed, jnp.int32)


def _fold_table(emb_t, w_t, b2d):
    D, V = emb_t.shape
    C = w_t.shape[0]
    RB = 4096
    return pl.pallas_call(
        _fold_body,
        grid=(pl.cdiv(V, RB),),
        in_specs=[
            pl.BlockSpec((D, RB), lambda i: (0, i)),
            pl.BlockSpec((C, D), lambda i: (0, 0)),
            pl.BlockSpec((C, 1), lambda i: (0, 0)),
        ],
        out_specs=pl.BlockSpec((1, RB), lambda i: (0, i)),
        out_shape=jax.ShapeDtypeStruct((1, V), jnp.int32),
    )(emb_t, w_t, b2d)


def _make_bag_kernel(V, C, B, L):
    assert C == 2 and B % (_NUM_WORKERS * _LANES) == 0 and L % 8 == 0
    bags_per_worker = B // _NUM_WORKERS
    n_chunks = 4
    chunk_bags = bags_per_worker // n_chunks
    groups_per_chunk = chunk_bags // _LANES
    inv_l = 1.0 / float(L)

    mesh = plsc.VectorSubcoreMesh(core_axis_name="c", subcore_axis_name="s")

    @functools.partial(
        pl.kernel,
        out_type=jax.ShapeDtypeStruct((C, B), jnp.float32),
        mesh=mesh,
        scratch_types=[
            pltpu.VMEM((V,), jnp.int32),
            pltpu.VMEM((L, chunk_bags), jnp.int32),
            pltpu.VMEM((L, chunk_bags), jnp.int32),
            pltpu.VMEM((C * bags_per_worker,), jnp.float32),
            pltpu.SemaphoreType.DMA,
            pltpu.SemaphoreType.DMA,
            pltpu.SemaphoreType.DMA,
        ],
        compiler_params=pltpu.CompilerParams(needs_layout_passes=False),
    )
    def bag_kernel(a_hbm, t_hbm, o_hbm, a_v, t_v0, t_v1, o_v, sem0, sem1, sem_a):
        wid = lax.axis_index("s") * _NUM_CORES + lax.axis_index("c")
        base = wid * bags_per_worker

        a_cp = pltpu.async_copy(a_hbm.at[0, :], a_v, sem_a)

        t_bufs = (t_v0, t_v1)
        sems = (sem0, sem1)

        def start_chunk(ci):
            src = t_hbm.at[:, pl.ds(base + ci * chunk_bags, chunk_bags)]
            return pltpu.async_copy(src, t_bufs[ci % 2], sems[ci % 2])

        pending = start_chunk(0)

        himask = jnp.full((_LANES,), -65536, jnp.int32)
        zacc = jnp.zeros((_LANES,), jnp.float32)

        a_cp.wait()

        for ci in range(n_chunks):
            pending.wait()
            if ci + 1 < n_chunks:
                pending = start_chunk(ci + 1)
            tbuf = t_bufs[ci % 2]

            def group(g, carry, tbuf=tbuf, ci=ci):
                col0 = g * _LANES

                def step(t, accs):
                    acc0, acc1 = accs
                    idx = tbuf[t, pl.ds(col0, _LANES)]
                    w = plsc.load_gather(a_v, [idx])
                    lo = plsc.bitcast(w << 16, jnp.float32)
                    hi = plsc.bitcast(w & himask, jnp.float32)
                    return acc0 + lo, acc1 + hi

                acc0, acc1 = lax.fori_loop(0, L, step, (zacc, zacc), unroll=10)
                obag = (ci * groups_per_chunk + g) * _LANES + lax.iota(jnp.int32, _LANES)
                plsc.store_scatter(o_v, [obag], acc0 * inv_l)
                plsc.store_scatter(o_v, [obag + bags_per_worker], acc1 * inv_l)
                return carry

            lax.fori_loop(0, groups_per_chunk, group, 0)

        pltpu.sync_copy(o_v.at[pl.ds(0, bags_per_worker)],
                        o_hbm.at[0, pl.ds(base, bags_per_worker)])
        pltpu.sync_copy(o_v.at[pl.ds(bags_per_worker, bags_per_worker)],
                        o_hbm.at[1, pl.ds(base, bags_per_worker)])

    return bag_kernel


def kernel(texts, emb_table, W, b):
    V, D = emb_table.shape
    C = W.shape[1]
    B, L = texts.shape
    A = _fold_table(emb_table.T, W.T, b.reshape(C, 1))
    bag = _make_bag_kernel(V, C, B, L)
    out_planar = bag(A, texts.astype(jnp.int32).T)
    return out_planar.T
